# baseline (speedup 1.0000x reference)
import functools

import jax
import jax.numpy as jnp
from jax import lax
from jax.experimental import pallas as pl
from jax.experimental.pallas import tpu as pltpu
from jax.experimental.pallas import tpu_sc as plsc

NUM_ENT = 1000000
ENT_DIM = 128
HIDDEN_DIM = 256
NUM_LAYERS = 2
B = 4096

_NC = 2
_NS = 16
_NW = _NC * _NS
_BPW = B // _NW

@functools.lru_cache(maxsize=None)
def _make_sc_kernels():
    mesh = plsc.VectorSubcoreMesh(
        core_axis_name="c", subcore_axis_name="s", num_cores=_NC, num_subcores=_NS
    )

    @functools.partial(
        pl.kernel,
        mesh=mesh,
        out_type=jax.ShapeDtypeStruct((B, ENT_DIM), jnp.float32),
        scratch_types=[
            pltpu.VMEM((_BPW,), jnp.int32),
            pltpu.VMEM((_BPW, ENT_DIM), jnp.float32),
            pltpu.SemaphoreType.DMA,
        ],
    )
    def sc_gather(table_hbm, idx_hbm, out_hbm, idx_v, rows_v, sem):
        wid = lax.axis_index("s") * _NC + lax.axis_index("c")
        base = wid * _BPW
        pltpu.sync_copy(idx_hbm.at[pl.ds(base, _BPW)], idx_v)
        pltpu.async_copy(table_hbm.at[idx_v], rows_v, sem).wait()
        pltpu.sync_copy(rows_v, out_hbm.at[pl.ds(base, _BPW)])

    _ROWS_PER_W = (NUM_ENT // _NW) // 8 * 8
    _TAIL = NUM_ENT - _NW * _ROWS_PER_W

    @functools.partial(
        pl.kernel,
        mesh=mesh,
        out_type=jax.ShapeDtypeStruct((NUM_ENT, ENT_DIM), jnp.float32),
        scratch_types=[pltpu.SemaphoreType.DMA],
    )
    def sc_copy(table_hbm, out_hbm, sem):
        wid = lax.axis_index("s") * _NC + lax.axis_index("c")
        sl = pl.ds(wid * _ROWS_PER_W, _ROWS_PER_W)
        pltpu.async_copy(table_hbm.at[sl], out_hbm.at[sl], sem).wait()

        @pl.when(wid == _NW - 1)
        def _():
            tl = pl.ds(_NW * _ROWS_PER_W, _TAIL)
            pltpu.async_copy(table_hbm.at[tl], out_hbm.at[tl], sem).wait()

    @functools.partial(
        pl.kernel,
        mesh=mesh,
        out_type=(),
        scratch_types=[
            pltpu.VMEM((_BPW,), jnp.int32),
            pltpu.VMEM((_BPW, ENT_DIM), jnp.float32),
            pltpu.SemaphoreType.DMA,
        ],
    )
    def sc_scatter(rows_hbm, idx_hbm, table_ref, idx_v, rows_v, sem):
        wid = lax.axis_index("s") * _NC + lax.axis_index("c")
        base = wid * _BPW
        pltpu.sync_copy(idx_hbm.at[pl.ds(base, _BPW)], idx_v)
        pltpu.sync_copy(rows_hbm.at[pl.ds(base, _BPW)], rows_v)
        pltpu.async_copy(rows_v, table_ref.at[idx_v], sem).wait()

    return sc_gather, sc_scatter, sc_copy


def _gate_body(x_ref, win_ref, prev_ref, gw_ref, gb_ref, mult_ref, out_ref):
    x = x_ref[...]
    norm = jnp.maximum(jnp.sum(jnp.abs(x), axis=-1, keepdims=True), 1e-12)
    e = jnp.dot(x / norm, win_ref[...], preferred_element_type=jnp.float32)
    prev = prev_ref[...] * mult_ref[...]
    g = jax.nn.sigmoid(
        jnp.dot(e, gw_ref[0], preferred_element_type=jnp.float32)
        + jnp.dot(prev, gw_ref[1], preferred_element_type=jnp.float32)
        + gb_ref[...]
    )
    out_ref[...] = g * e + (1.0 - g) * prev


def _tc_gate(x, win, prev, gw2, gb, mult):
    return pl.pallas_call(
        _gate_body,
        out_shape=jax.ShapeDtypeStruct((B, ENT_DIM), jnp.float32),
        compiler_params=pltpu.CompilerParams(vmem_limit_bytes=100 * 1024 * 1024),
    )(x, win, prev, gw2, gb, mult)


_BR = 512


def _layer_body(adj_ref, h_ref, w1_ref, w2_ref, out_ref):
    i = pl.program_id(0)
    a = adj_ref[...]
    deg = jnp.sum(a, axis=-1, keepdims=True) + 1e-6
    m = jnp.dot(a, h_ref[...], preferred_element_type=jnp.float32) / deg
    z = jnp.maximum(jnp.dot(m, w1_ref[...], preferred_element_type=jnp.float32), 0.0)
    out_ref[...] = h_ref[pl.ds(i * _BR, _BR), :] + jnp.dot(
        z, w2_ref[...], preferred_element_type=jnp.float32
    )


def _tc_layer(adj, h, w1, w2):
    return pl.pallas_call(
        _layer_body,
        out_shape=jax.ShapeDtypeStruct((B, ENT_DIM), jnp.float32),
        grid=(B // _BR,),
        in_specs=[
            pl.BlockSpec((_BR, B), lambda i: (i, 0)),
            pl.BlockSpec((B, ENT_DIM), lambda i: (0, 0)),
            pl.BlockSpec((ENT_DIM, HIDDEN_DIM), lambda i: (0, 0)),
            pl.BlockSpec((HIDDEN_DIM, ENT_DIM), lambda i: (0, 0)),
        ],
        out_specs=pl.BlockSpec((_BR, ENT_DIM), lambda i: (i, 0)),
        compiler_params=pltpu.CompilerParams(
            dimension_semantics=("arbitrary",),
            vmem_limit_bytes=100 * 1024 * 1024,
        ),
    )(adj, h, w1, w2)


def kernel(ent_relational_fearues, batch_ent_idxs, relational_adj_matrices,
           re_ratio, memory_cells, Win_W, gate_W, gate_b, layer_W1, layer_W2):
    idxs = batch_ent_idxs.astype(jnp.int32)

    n = B
    num_re = jnp.floor(n * jnp.asarray(re_ratio, dtype=jnp.float32)).astype(jnp.int32)
    perm = jax.random.permutation(jax.random.key(42), n)
    keep = jnp.where(jnp.arange(n) < num_re, 0.0, 1.0).astype(jnp.float32)
    row_mult = jnp.ones((n,), dtype=jnp.float32).at[perm].set(keep)
    mult = row_mult[:, None]

    sc_gather, sc_scatter, sc_copy = _make_sc_kernels()
    prev = sc_gather(memory_cells, idxs)

    gw2 = gate_W.reshape(2, ENT_DIM, ENT_DIM)
    memory_out = _tc_gate(ent_relational_fearues, Win_W, prev, gw2, gate_b, mult)

    copied = sc_copy(memory_cells)
    table_ref = jax.new_ref(copied)
    sc_scatter(memory_out, idxs, table_ref)
    new_memory_cells = jax.freeze(table_ref)

    h = memory_out
    for i in range(NUM_LAYERS):
        h = _tc_layer(relational_adj_matrices, h, layer_W1[i], layer_W2[i])
    return h, new_memory_cells


# speedup vs baseline: 34.4650x; 34.4650x over previous
import functools

import jax
import jax.numpy as jnp
from jax import lax
from jax.experimental import pallas as pl
from jax.experimental.pallas import tpu as pltpu
from jax.experimental.pallas import tpu_sc as plsc

NUM_ENT = 1000000
ENT_DIM = 128
HIDDEN_DIM = 256
NUM_LAYERS = 2
B = 4096

_NC = 2
_NS = 16
_NW = _NC * _NS
_BPW = B // _NW

@functools.lru_cache(maxsize=None)
def _make_sc_kernels():
    mesh = plsc.VectorSubcoreMesh(
        core_axis_name="c", subcore_axis_name="s", num_cores=_NC, num_subcores=_NS
    )

    @functools.partial(
        pl.kernel,
        mesh=mesh,
        out_type=jax.ShapeDtypeStruct((B, ENT_DIM), jnp.float32),
        scratch_types=[
            pltpu.VMEM((_BPW,), jnp.int32),
            pltpu.VMEM((_BPW, ENT_DIM), jnp.float32),
            pltpu.SemaphoreType.DMA,
        ],
    )
    def sc_gather(table_hbm, idx_hbm, out_hbm, idx_v, rows_v, sem):
        wid = lax.axis_index("s") * _NC + lax.axis_index("c")
        base = wid * _BPW
        pltpu.sync_copy(idx_hbm.at[pl.ds(base, _BPW)], idx_v)
        pltpu.async_copy(table_hbm.at[idx_v], rows_v, sem).wait()
        pltpu.sync_copy(rows_v, out_hbm.at[pl.ds(base, _BPW)])

    _CH = 256
    _ROWS_PER_W = 31232
    _PAIRS = (_ROWS_PER_W // _CH) // 2
    _LAST_ROWS = NUM_ENT - (_NW - 1) * _ROWS_PER_W
    _LAST_PAIRS = 62
    _TAIL = _LAST_ROWS - 2 * _LAST_PAIRS * _CH

    @functools.partial(
        pl.kernel,
        mesh=mesh,
        out_type=jax.ShapeDtypeStruct((NUM_ENT, ENT_DIM), jnp.float32),
        scratch_types=[
            pltpu.VMEM((_CH, ENT_DIM), jnp.float32),
            pltpu.VMEM((_CH, ENT_DIM), jnp.float32),
            pltpu.VMEM((_TAIL, ENT_DIM), jnp.float32),
            pltpu.SemaphoreType.DMA,
            pltpu.SemaphoreType.DMA,
            pltpu.SemaphoreType.DMA,
            pltpu.SemaphoreType.DMA,
        ],
    )
    def sc_copy(table_hbm, out_hbm, buf_a, buf_b, buf_t, rs_a, rs_b, ws_a, ws_b):
        wid = lax.axis_index("s") * _NC + lax.axis_index("c")
        base = pl.multiple_of(wid * _ROWS_PER_W, 8)
        n_pairs = jnp.where(wid == _NW - 1, _LAST_PAIRS, _PAIRS)

        def pair(j, _):
            o_a = pl.multiple_of(base + (2 * j) * _CH, 8)
            o_b = pl.multiple_of(base + (2 * j + 1) * _CH, 8)
            ra = pltpu.make_async_copy(table_hbm.at[pl.ds(o_a, _CH)], buf_a, rs_a)
            rb = pltpu.make_async_copy(table_hbm.at[pl.ds(o_b, _CH)], buf_b, rs_b)
            ra.start()
            rb.start()
            ra.wait()
            wa = pltpu.make_async_copy(buf_a, out_hbm.at[pl.ds(o_a, _CH)], ws_a)
            wa.start()
            rb.wait()
            wb = pltpu.make_async_copy(buf_b, out_hbm.at[pl.ds(o_b, _CH)], ws_b)
            wb.start()
            wa.wait()
            wb.wait()
            return _

        lax.fori_loop(0, n_pairs, pair, 0)

        @pl.when(wid == _NW - 1)
        def _():
            tl = pl.ds(NUM_ENT - _TAIL, _TAIL)
            pltpu.sync_copy(table_hbm.at[tl], buf_t)
            pltpu.sync_copy(buf_t, out_hbm.at[tl])

    @functools.partial(
        pl.kernel,
        mesh=mesh,
        out_type=(),
        scratch_types=[
            pltpu.VMEM((_BPW,), jnp.int32),
            pltpu.VMEM((_BPW, ENT_DIM), jnp.float32),
            pltpu.SemaphoreType.DMA,
        ],
    )
    def sc_scatter(rows_hbm, idx_hbm, table_ref, idx_v, rows_v, sem):
        wid = lax.axis_index("s") * _NC + lax.axis_index("c")
        base = wid * _BPW
        pltpu.sync_copy(idx_hbm.at[pl.ds(base, _BPW)], idx_v)
        pltpu.sync_copy(rows_hbm.at[pl.ds(base, _BPW)], rows_v)
        pltpu.async_copy(rows_v, table_ref.at[idx_v], sem).wait()

    return sc_gather, sc_scatter, sc_copy


def _gate_body(x_ref, win_ref, prev_ref, gw_ref, gb_ref, mult_ref, out_ref):
    x = x_ref[...]
    norm = jnp.maximum(jnp.sum(jnp.abs(x), axis=-1, keepdims=True), 1e-12)
    e = jnp.dot(x / norm, win_ref[...], preferred_element_type=jnp.float32)
    prev = prev_ref[...] * mult_ref[...]
    g = jax.nn.sigmoid(
        jnp.dot(e, gw_ref[0], preferred_element_type=jnp.float32)
        + jnp.dot(prev, gw_ref[1], preferred_element_type=jnp.float32)
        + gb_ref[...]
    )
    out_ref[...] = g * e + (1.0 - g) * prev


def _tc_gate(x, win, prev, gw2, gb, mult):
    return pl.pallas_call(
        _gate_body,
        out_shape=jax.ShapeDtypeStruct((B, ENT_DIM), jnp.float32),
        compiler_params=pltpu.CompilerParams(vmem_limit_bytes=100 * 1024 * 1024),
    )(x, win, prev, gw2, gb, mult)


_BR = 512


def _layer_body(adj_ref, h_ref, w1_ref, w2_ref, out_ref):
    i = pl.program_id(0)
    a = adj_ref[...]
    deg = jnp.sum(a, axis=-1, keepdims=True) + 1e-6
    m = jnp.dot(a, h_ref[...], preferred_element_type=jnp.float32) / deg
    z = jnp.maximum(jnp.dot(m, w1_ref[...], preferred_element_type=jnp.float32), 0.0)
    out_ref[...] = h_ref[pl.ds(i * _BR, _BR), :] + jnp.dot(
        z, w2_ref[...], preferred_element_type=jnp.float32
    )


def _tc_layer(adj, h, w1, w2):
    return pl.pallas_call(
        _layer_body,
        out_shape=jax.ShapeDtypeStruct((B, ENT_DIM), jnp.float32),
        grid=(B // _BR,),
        in_specs=[
            pl.BlockSpec((_BR, B), lambda i: (i, 0)),
            pl.BlockSpec((B, ENT_DIM), lambda i: (0, 0)),
            pl.BlockSpec((ENT_DIM, HIDDEN_DIM), lambda i: (0, 0)),
            pl.BlockSpec((HIDDEN_DIM, ENT_DIM), lambda i: (0, 0)),
        ],
        out_specs=pl.BlockSpec((_BR, ENT_DIM), lambda i: (i, 0)),
        compiler_params=pltpu.CompilerParams(
            dimension_semantics=("arbitrary",),
            vmem_limit_bytes=100 * 1024 * 1024,
        ),
    )(adj, h, w1, w2)


def kernel(ent_relational_fearues, batch_ent_idxs, relational_adj_matrices,
           re_ratio, memory_cells, Win_W, gate_W, gate_b, layer_W1, layer_W2):
    idxs = batch_ent_idxs.astype(jnp.int32)

    n = B
    num_re = jnp.floor(n * jnp.asarray(re_ratio, dtype=jnp.float32)).astype(jnp.int32)
    perm = jax.random.permutation(jax.random.key(42), n)
    keep = jnp.where(jnp.arange(n) < num_re, 0.0, 1.0).astype(jnp.float32)
    row_mult = jnp.ones((n,), dtype=jnp.float32).at[perm].set(keep)
    mult = row_mult[:, None]

    sc_gather, sc_scatter, sc_copy = _make_sc_kernels()
    prev = sc_gather(memory_cells, idxs)

    gw2 = gate_W.reshape(2, ENT_DIM, ENT_DIM)
    memory_out = _tc_gate(ent_relational_fearues, Win_W, prev, gw2, gate_b, mult)

    copied = sc_copy(memory_cells)
    table_ref = jax.new_ref(copied)
    sc_scatter(memory_out, idxs, table_ref)
    new_memory_cells = jax.freeze(table_ref)

    h = memory_out
    for i in range(NUM_LAYERS):
        h = _tc_layer(relational_adj_matrices, h, layer_W1[i], layer_W2[i])
    return h, new_memory_cells


# speedup vs baseline: 34.5158x; 1.0015x over previous
import functools

import jax
import jax.numpy as jnp
from jax import lax
from jax.experimental import pallas as pl
from jax.experimental.pallas import tpu as pltpu
from jax.experimental.pallas import tpu_sc as plsc

NUM_ENT = 1000000
ENT_DIM = 128
HIDDEN_DIM = 256
NUM_LAYERS = 2
B = 4096

_NC = 2
_NS = 16
_NW = _NC * _NS
_BPW = B // _NW

@functools.lru_cache(maxsize=None)
def _make_sc_kernels():
    mesh = plsc.VectorSubcoreMesh(
        core_axis_name="c", subcore_axis_name="s", num_cores=_NC, num_subcores=_NS
    )

    @functools.partial(
        pl.kernel,
        mesh=mesh,
        out_type=jax.ShapeDtypeStruct((B, ENT_DIM), jnp.float32),
        scratch_types=[
            pltpu.VMEM((_BPW,), jnp.int32),
            pltpu.VMEM((_BPW, ENT_DIM), jnp.float32),
            pltpu.SemaphoreType.DMA,
        ],
    )
    def sc_gather(table_hbm, idx_hbm, out_hbm, idx_v, rows_v, sem):
        wid = lax.axis_index("s") * _NC + lax.axis_index("c")
        base = wid * _BPW
        pltpu.sync_copy(idx_hbm.at[pl.ds(base, _BPW)], idx_v)
        pltpu.async_copy(table_hbm.at[idx_v], rows_v, sem).wait()
        pltpu.sync_copy(rows_v, out_hbm.at[pl.ds(base, _BPW)])

    _CH = 256
    _ROWS_PER_W = 31232
    _PAIRS = (_ROWS_PER_W // _CH) // 2
    _LAST_ROWS = NUM_ENT - (_NW - 1) * _ROWS_PER_W
    _LAST_PAIRS = 62
    _TAIL = _LAST_ROWS - 2 * _LAST_PAIRS * _CH

    @functools.partial(
        pl.kernel,
        mesh=mesh,
        out_type=jax.ShapeDtypeStruct((NUM_ENT, ENT_DIM), jnp.float32),
        scratch_types=[
            pltpu.VMEM((_CH, ENT_DIM), jnp.float32),
            pltpu.VMEM((_CH, ENT_DIM), jnp.float32),
            pltpu.VMEM((_TAIL, ENT_DIM), jnp.float32),
            pltpu.SemaphoreType.DMA,
            pltpu.SemaphoreType.DMA,
            pltpu.SemaphoreType.DMA,
            pltpu.SemaphoreType.DMA,
        ],
    )
    def sc_copy(table_hbm, out_hbm, buf_a, buf_b, buf_t, rs_a, rs_b, ws_a, ws_b):
        wid = lax.axis_index("s") * _NC + lax.axis_index("c")
        base = pl.multiple_of(wid * _ROWS_PER_W, 8)
        n_pairs = jnp.where(wid == _NW - 1, _LAST_PAIRS, _PAIRS)

        def pair(j, _):
            o_a = pl.multiple_of(base + (2 * j) * _CH, 8)
            o_b = pl.multiple_of(base + (2 * j + 1) * _CH, 8)
            ra = pltpu.make_async_copy(table_hbm.at[pl.ds(o_a, _CH)], buf_a, rs_a)
            rb = pltpu.make_async_copy(table_hbm.at[pl.ds(o_b, _CH)], buf_b, rs_b)
            ra.start()
            rb.start()
            ra.wait()
            wa = pltpu.make_async_copy(buf_a, out_hbm.at[pl.ds(o_a, _CH)], ws_a)
            wa.start()
            rb.wait()
            wb = pltpu.make_async_copy(buf_b, out_hbm.at[pl.ds(o_b, _CH)], ws_b)
            wb.start()
            wa.wait()
            wb.wait()
            return _

        lax.fori_loop(0, n_pairs, pair, 0)

        @pl.when(wid == _NW - 1)
        def _():
            tl = pl.ds(NUM_ENT - _TAIL, _TAIL)
            pltpu.sync_copy(table_hbm.at[tl], buf_t)
            pltpu.sync_copy(buf_t, out_hbm.at[tl])

    @functools.partial(
        pl.kernel,
        mesh=mesh,
        out_type=(),
        scratch_types=[
            pltpu.VMEM((_BPW,), jnp.int32),
            pltpu.VMEM((_BPW, ENT_DIM), jnp.float32),
            pltpu.SemaphoreType.DMA,
        ],
    )
    def sc_scatter(rows_hbm, idx_hbm, table_ref, idx_v, rows_v, sem):
        wid = lax.axis_index("s") * _NC + lax.axis_index("c")
        base = wid * _BPW
        pltpu.sync_copy(idx_hbm.at[pl.ds(base, _BPW)], idx_v)
        pltpu.sync_copy(rows_hbm.at[pl.ds(base, _BPW)], rows_v)
        pltpu.async_copy(rows_v, table_ref.at[idx_v], sem).wait()

    return sc_gather, sc_scatter, sc_copy


def _gate_body(x_ref, win_ref, prev_ref, gw_ref, gb_ref, mult_ref, out_ref):
    x = x_ref[...]
    norm = jnp.maximum(jnp.sum(jnp.abs(x), axis=-1, keepdims=True), 1e-12)
    e = jnp.dot(x / norm, win_ref[...], preferred_element_type=jnp.float32)
    prev = prev_ref[...] * mult_ref[...]
    g = jax.nn.sigmoid(
        jnp.dot(e, gw_ref[0], preferred_element_type=jnp.float32)
        + jnp.dot(prev, gw_ref[1], preferred_element_type=jnp.float32)
        + gb_ref[...]
    )
    out_ref[...] = g * e + (1.0 - g) * prev


def _tc_gate(x, win, prev, gw2, gb, mult):
    return pl.pallas_call(
        _gate_body,
        out_shape=jax.ShapeDtypeStruct((B, ENT_DIM), jnp.float32),
        compiler_params=pltpu.CompilerParams(vmem_limit_bytes=100 * 1024 * 1024),
    )(x, win, prev, gw2, gb, mult)


_BR = 512


def _layer_body(adj_ref, h_ref, w1_ref, w2_ref, out_ref):
    i = pl.program_id(0)
    a = adj_ref[...]
    deg = jnp.sum(a, axis=-1, keepdims=True) + 1e-6
    m = jnp.dot(a, h_ref[...], preferred_element_type=jnp.float32) / deg
    z = jnp.maximum(jnp.dot(m, w1_ref[...], preferred_element_type=jnp.float32), 0.0)
    out_ref[...] = h_ref[pl.ds(i * _BR, _BR), :] + jnp.dot(
        z, w2_ref[...], preferred_element_type=jnp.float32
    )


def _tc_layer(adj, h, w1, w2):
    return pl.pallas_call(
        _layer_body,
        out_shape=jax.ShapeDtypeStruct((B, ENT_DIM), jnp.float32),
        grid=(B // _BR,),
        in_specs=[
            pl.BlockSpec((_BR, B), lambda i: (i, 0)),
            pl.BlockSpec((B, ENT_DIM), lambda i: (0, 0)),
            pl.BlockSpec((ENT_DIM, HIDDEN_DIM), lambda i: (0, 0)),
            pl.BlockSpec((HIDDEN_DIM, ENT_DIM), lambda i: (0, 0)),
        ],
        out_specs=pl.BlockSpec((_BR, ENT_DIM), lambda i: (i, 0)),
        compiler_params=pltpu.CompilerParams(
            dimension_semantics=("arbitrary",),
            vmem_limit_bytes=100 * 1024 * 1024,
        ),
    )(adj, h, w1, w2)


def kernel(ent_relational_fearues, batch_ent_idxs, relational_adj_matrices,
           re_ratio, memory_cells, Win_W, gate_W, gate_b, layer_W1, layer_W2):
    idxs = batch_ent_idxs.astype(jnp.int32)

    n = B
    num_re = jnp.floor(n * jnp.asarray(re_ratio, dtype=jnp.float32)).astype(jnp.int32)
    perm = jax.random.permutation(jax.random.key(42), n)
    keep = jnp.where(jnp.arange(n) < num_re, 0.0, 1.0).astype(jnp.float32)
    row_mult = jnp.ones((n,), dtype=jnp.float32).at[perm].set(keep)
    mult = row_mult[:, None]

    sc_gather, sc_scatter, sc_copy = _make_sc_kernels()
    prev = sc_gather(memory_cells, idxs)

    gw2 = gate_W.reshape(2, ENT_DIM, ENT_DIM)
    memory_out = _tc_gate(ent_relational_fearues, Win_W, prev, gw2, gate_b, mult)

    copied = sc_copy(memory_cells)

    h = memory_out
    for i in range(NUM_LAYERS):
        h = _tc_layer(relational_adj_matrices, h, layer_W1[i], layer_W2[i])

    table_ref = jax.new_ref(copied)
    sc_scatter(memory_out, idxs, table_ref)
    new_memory_cells = jax.freeze(table_ref)
    return h, new_memory_cells


# speedup vs baseline: 35.6800x; 1.0337x over previous
import functools

import jax
import jax.numpy as jnp
from jax import lax
from jax.experimental import pallas as pl
from jax.experimental.pallas import tpu as pltpu
from jax.experimental.pallas import tpu_sc as plsc

NUM_ENT = 1000000
ENT_DIM = 128
HIDDEN_DIM = 256
NUM_LAYERS = 2
B = 4096

_NC = 2
_NS = 16
_NW = _NC * _NS
_BPW = B // _NW

@functools.lru_cache(maxsize=None)
def _make_sc_kernels():
    mesh = plsc.VectorSubcoreMesh(
        core_axis_name="c", subcore_axis_name="s", num_cores=_NC, num_subcores=_NS
    )

    @functools.partial(
        pl.kernel,
        mesh=mesh,
        out_type=jax.ShapeDtypeStruct((B, ENT_DIM), jnp.float32),
        scratch_types=[
            pltpu.VMEM((_BPW,), jnp.int32),
            pltpu.VMEM((_BPW, ENT_DIM), jnp.float32),
            pltpu.SemaphoreType.DMA,
        ],
    )
    def sc_gather(table_hbm, idx_hbm, out_hbm, idx_v, rows_v, sem):
        wid = lax.axis_index("s") * _NC + lax.axis_index("c")
        base = wid * _BPW
        pltpu.sync_copy(idx_hbm.at[pl.ds(base, _BPW)], idx_v)
        pltpu.async_copy(table_hbm.at[idx_v], rows_v, sem).wait()
        pltpu.sync_copy(rows_v, out_hbm.at[pl.ds(base, _BPW)])

    _CH = 256
    _ROWS_PER_W = 31232
    _PAIRS = (_ROWS_PER_W // _CH) // 2
    _LAST_ROWS = NUM_ENT - (_NW - 1) * _ROWS_PER_W
    _LAST_PAIRS = 62
    _TAIL = _LAST_ROWS - 2 * _LAST_PAIRS * _CH

    @functools.partial(
        pl.kernel,
        mesh=mesh,
        out_type=jax.ShapeDtypeStruct((NUM_ENT, ENT_DIM), jnp.float32),
        scratch_types=[
            pltpu.VMEM((_CH, ENT_DIM), jnp.float32),
            pltpu.VMEM((_CH, ENT_DIM), jnp.float32),
            pltpu.VMEM((_TAIL, ENT_DIM), jnp.float32),
            pltpu.SemaphoreType.DMA,
            pltpu.SemaphoreType.DMA,
            pltpu.SemaphoreType.DMA,
            pltpu.SemaphoreType.DMA,
        ],
    )
    def sc_copy(table_hbm, out_hbm, buf_a, buf_b, buf_t, rs_a, rs_b, ws_a, ws_b):
        wid = lax.axis_index("s") * _NC + lax.axis_index("c")
        base = pl.multiple_of(wid * _ROWS_PER_W, 8)
        n_pairs = jnp.where(wid == _NW - 1, _LAST_PAIRS, _PAIRS)

        def pair(j, _):
            o_a = pl.multiple_of(base + (2 * j) * _CH, 8)
            o_b = pl.multiple_of(base + (2 * j + 1) * _CH, 8)
            ra = pltpu.make_async_copy(table_hbm.at[pl.ds(o_a, _CH)], buf_a, rs_a)
            rb = pltpu.make_async_copy(table_hbm.at[pl.ds(o_b, _CH)], buf_b, rs_b)
            ra.start()
            rb.start()
            ra.wait()
            wa = pltpu.make_async_copy(buf_a, out_hbm.at[pl.ds(o_a, _CH)], ws_a)
            wa.start()
            rb.wait()
            wb = pltpu.make_async_copy(buf_b, out_hbm.at[pl.ds(o_b, _CH)], ws_b)
            wb.start()
            wa.wait()
            wb.wait()
            return _

        lax.fori_loop(0, n_pairs, pair, 0)

        @pl.when(wid == _NW - 1)
        def _():
            tl = pl.ds(NUM_ENT - _TAIL, _TAIL)
            pltpu.sync_copy(table_hbm.at[tl], buf_t)
            pltpu.sync_copy(buf_t, out_hbm.at[tl])

    @functools.partial(
        pl.kernel,
        mesh=mesh,
        out_type=(),
        scratch_types=[
            pltpu.VMEM((_BPW,), jnp.int32),
            pltpu.VMEM((_BPW, ENT_DIM), jnp.float32),
            pltpu.SemaphoreType.DMA,
        ],
    )
    def sc_scatter(rows_hbm, idx_hbm, table_ref, idx_v, rows_v, sem):
        wid = lax.axis_index("s") * _NC + lax.axis_index("c")
        base = wid * _BPW
        pltpu.sync_copy(idx_hbm.at[pl.ds(base, _BPW)], idx_v)
        pltpu.sync_copy(rows_hbm.at[pl.ds(base, _BPW)], rows_v)
        pltpu.async_copy(rows_v, table_ref.at[idx_v], sem).wait()

    return sc_gather, sc_scatter, sc_copy


def _gate_body(x_ref, win_ref, prev_ref, gw_ref, gb_ref, mult_ref, out_ref):
    x = x_ref[...]
    norm = jnp.maximum(jnp.sum(jnp.abs(x), axis=-1, keepdims=True), 1e-12)
    e = jnp.dot(x / norm, win_ref[...], preferred_element_type=jnp.float32)
    prev = prev_ref[...] * mult_ref[...]
    g = jax.nn.sigmoid(
        jnp.dot(e, gw_ref[0], preferred_element_type=jnp.float32)
        + jnp.dot(prev, gw_ref[1], preferred_element_type=jnp.float32)
        + gb_ref[...]
    )
    out_ref[...] = g * e + (1.0 - g) * prev


def _tc_gate(x, win, prev, gw2, gb, mult):
    return pl.pallas_call(
        _gate_body,
        out_shape=jax.ShapeDtypeStruct((B, ENT_DIM), jnp.float32),
        compiler_params=pltpu.CompilerParams(vmem_limit_bytes=100 * 1024 * 1024),
    )(x, win, prev, gw2, gb, mult)


_BR = 512
_NBLK = B // _BR


def _layers_body(adj_ref, h0_ref, w1_ref, w2_ref, out_ref,
                 abf_ref, h1_ref, rinv_ref):
    i = pl.program_id(0)

    @pl.when(i < _NBLK)
    def _():
        a = adj_ref[...]
        abf_ref[pl.ds(i * _BR, _BR), :] = a.astype(jnp.bfloat16)
        rinv = 1.0 / (jnp.sum(a, axis=-1, keepdims=True) + 1e-6)
        rinv_ref[pl.ds(i * _BR, _BR), :] = rinv
        m = jnp.dot(a, h0_ref[...], preferred_element_type=jnp.float32) * rinv
        z = jnp.maximum(
            jnp.dot(m, w1_ref[0], preferred_element_type=jnp.float32), 0.0)
        h1_ref[pl.ds(i * _BR, _BR), :] = h0_ref[pl.ds(i * _BR, _BR), :] + jnp.dot(
            z, w2_ref[0], preferred_element_type=jnp.float32)

    @pl.when(i >= _NBLK)
    def _():
        j = i - _NBLK
        a16 = abf_ref[pl.ds(j * _BR, _BR), :]
        m = jnp.dot(a16, h1_ref[...].astype(jnp.bfloat16),
                    preferred_element_type=jnp.float32)
        m = m * rinv_ref[pl.ds(j * _BR, _BR), :]
        z = jnp.maximum(
            jnp.dot(m, w1_ref[1], preferred_element_type=jnp.float32), 0.0)
        out_ref[...] = h1_ref[pl.ds(j * _BR, _BR), :] + jnp.dot(
            z, w2_ref[1], preferred_element_type=jnp.float32)


def _tc_layers(adj, h, w1, w2):
    return pl.pallas_call(
        _layers_body,
        out_shape=jax.ShapeDtypeStruct((B, ENT_DIM), jnp.float32),
        grid=(2 * _NBLK,),
        in_specs=[
            pl.BlockSpec((_BR, B), lambda i: (jnp.minimum(i, _NBLK - 1), 0)),
            pl.BlockSpec((B, ENT_DIM), lambda i: (0, 0)),
            pl.BlockSpec((NUM_LAYERS, ENT_DIM, HIDDEN_DIM), lambda i: (0, 0, 0)),
            pl.BlockSpec((NUM_LAYERS, HIDDEN_DIM, ENT_DIM), lambda i: (0, 0, 0)),
        ],
        out_specs=pl.BlockSpec(
            (_BR, ENT_DIM), lambda i: (jnp.maximum(i - _NBLK, 0), 0)),
        scratch_shapes=[
            pltpu.VMEM((B, B), jnp.bfloat16),
            pltpu.VMEM((B, ENT_DIM), jnp.float32),
            pltpu.VMEM((B, 1), jnp.float32),
        ],
        compiler_params=pltpu.CompilerParams(
            dimension_semantics=("arbitrary",),
            vmem_limit_bytes=62 * 1024 * 1024,
        ),
    )(adj, h, w1, w2)


def kernel(ent_relational_fearues, batch_ent_idxs, relational_adj_matrices,
           re_ratio, memory_cells, Win_W, gate_W, gate_b, layer_W1, layer_W2):
    idxs = batch_ent_idxs.astype(jnp.int32)

    n = B
    num_re = jnp.floor(n * jnp.asarray(re_ratio, dtype=jnp.float32)).astype(jnp.int32)
    perm = jax.random.permutation(jax.random.key(42), n)
    keep = jnp.where(jnp.arange(n) < num_re, 0.0, 1.0).astype(jnp.float32)
    row_mult = jnp.ones((n,), dtype=jnp.float32).at[perm].set(keep)
    mult = row_mult[:, None]

    sc_gather, sc_scatter, sc_copy = _make_sc_kernels()
    prev = sc_gather(memory_cells, idxs)

    gw2 = gate_W.reshape(2, ENT_DIM, ENT_DIM)
    memory_out = _tc_gate(ent_relational_fearues, Win_W, prev, gw2, gate_b, mult)

    copied = sc_copy(memory_cells)

    h = _tc_layers(relational_adj_matrices, memory_out, layer_W1, layer_W2)

    table_ref = jax.new_ref(copied)
    sc_scatter(memory_out, idxs, table_ref)
    new_memory_cells = jax.freeze(table_ref)
    return h, new_memory_cells


# speedup vs baseline: 36.0793x; 1.0112x over previous
import functools

import jax
import jax.numpy as jnp
from jax import lax
from jax.experimental import pallas as pl
from jax.experimental.pallas import tpu as pltpu
from jax.experimental.pallas import tpu_sc as plsc

NUM_ENT = 1000000
ENT_DIM = 128
HIDDEN_DIM = 256
NUM_LAYERS = 2
B = 4096

_NC = 2
_NS = 16
_NW = _NC * _NS
_BPW = B // _NW

@functools.lru_cache(maxsize=None)
def _make_sc_kernels():
    mesh = plsc.VectorSubcoreMesh(
        core_axis_name="c", subcore_axis_name="s", num_cores=_NC, num_subcores=_NS
    )

    @functools.partial(
        pl.kernel,
        mesh=mesh,
        out_type=jax.ShapeDtypeStruct((B, ENT_DIM), jnp.float32),
        scratch_types=[
            pltpu.VMEM((_BPW,), jnp.int32),
            pltpu.VMEM((_BPW, ENT_DIM), jnp.float32),
            pltpu.SemaphoreType.DMA,
        ],
    )
    def sc_gather(table_hbm, idx_hbm, out_hbm, idx_v, rows_v, sem):
        wid = lax.axis_index("s") * _NC + lax.axis_index("c")
        base = wid * _BPW
        pltpu.sync_copy(idx_hbm.at[pl.ds(base, _BPW)], idx_v)
        pltpu.async_copy(table_hbm.at[idx_v], rows_v, sem).wait()
        pltpu.sync_copy(rows_v, out_hbm.at[pl.ds(base, _BPW)])

    _CH = 488
    _ROWS_PER_W = 31232
    _PAIRS = (_ROWS_PER_W // _CH) // 2
    _LAST_BASE = (_NW - 1) * _ROWS_PER_W + _ROWS_PER_W
    _TAIL1 = 488
    _TAIL2 = NUM_ENT - _LAST_BASE - _TAIL1

    @functools.partial(
        pl.kernel,
        mesh=mesh,
        out_type=jax.ShapeDtypeStruct((NUM_ENT, ENT_DIM), jnp.float32),
        scratch_types=[
            pltpu.VMEM((_CH, ENT_DIM), jnp.float32),
            pltpu.VMEM((_CH, ENT_DIM), jnp.float32),
            pltpu.SemaphoreType.DMA,
            pltpu.SemaphoreType.DMA,
            pltpu.SemaphoreType.DMA,
            pltpu.SemaphoreType.DMA,
        ],
    )
    def sc_copy(table_hbm, out_hbm, buf_a, buf_b, rs_a, rs_b, ws_a, ws_b):
        wid = lax.axis_index("s") * _NC + lax.axis_index("c")
        base = pl.multiple_of(wid * _ROWS_PER_W, 8)

        def pair(j, _):
            o_a = pl.multiple_of(base + (2 * j) * _CH, 8)
            o_b = pl.multiple_of(base + (2 * j + 1) * _CH, 8)
            ra = pltpu.make_async_copy(table_hbm.at[pl.ds(o_a, _CH)], buf_a, rs_a)
            rb = pltpu.make_async_copy(table_hbm.at[pl.ds(o_b, _CH)], buf_b, rs_b)
            ra.start()
            rb.start()
            ra.wait()
            wa = pltpu.make_async_copy(buf_a, out_hbm.at[pl.ds(o_a, _CH)], ws_a)
            wa.start()
            rb.wait()
            wb = pltpu.make_async_copy(buf_b, out_hbm.at[pl.ds(o_b, _CH)], ws_b)
            wb.start()
            wa.wait()
            wb.wait()
            return _

        lax.fori_loop(0, _PAIRS, pair, 0)

        @pl.when(wid == _NW - 1)
        def _():
            t1 = pl.ds(_LAST_BASE, _TAIL1)
            pltpu.sync_copy(table_hbm.at[t1], buf_a)
            pltpu.sync_copy(buf_a, out_hbm.at[t1])
            t2 = pl.ds(_LAST_BASE + _TAIL1, _TAIL2)
            pltpu.sync_copy(table_hbm.at[t2], buf_b.at[pl.ds(0, _TAIL2)])
            pltpu.sync_copy(buf_b.at[pl.ds(0, _TAIL2)], out_hbm.at[t2])

    @functools.partial(
        pl.kernel,
        mesh=mesh,
        out_type=(),
        scratch_types=[
            pltpu.VMEM((_BPW,), jnp.int32),
            pltpu.VMEM((_BPW, ENT_DIM), jnp.float32),
            pltpu.SemaphoreType.DMA,
        ],
    )
    def sc_scatter(rows_hbm, idx_hbm, table_ref, idx_v, rows_v, sem):
        wid = lax.axis_index("s") * _NC + lax.axis_index("c")
        base = wid * _BPW
        pltpu.sync_copy(idx_hbm.at[pl.ds(base, _BPW)], idx_v)
        pltpu.sync_copy(rows_hbm.at[pl.ds(base, _BPW)], rows_v)
        pltpu.async_copy(rows_v, table_ref.at[idx_v], sem).wait()

    return sc_gather, sc_scatter, sc_copy


def _gate_body(x_ref, win_ref, prev_ref, gw_ref, gb_ref, mult_ref, out_ref):
    x = x_ref[...]
    norm = jnp.maximum(jnp.sum(jnp.abs(x), axis=-1, keepdims=True), 1e-12)
    e = jnp.dot(x / norm, win_ref[...], preferred_element_type=jnp.float32)
    prev = prev_ref[...] * mult_ref[...]
    g = jax.nn.sigmoid(
        jnp.dot(e, gw_ref[0], preferred_element_type=jnp.float32)
        + jnp.dot(prev, gw_ref[1], preferred_element_type=jnp.float32)
        + gb_ref[...]
    )
    out_ref[...] = g * e + (1.0 - g) * prev


def _tc_gate(x, win, prev, gw2, gb, mult):
    return pl.pallas_call(
        _gate_body,
        out_shape=jax.ShapeDtypeStruct((B, ENT_DIM), jnp.float32),
        compiler_params=pltpu.CompilerParams(vmem_limit_bytes=100 * 1024 * 1024),
    )(x, win, prev, gw2, gb, mult)


_BR = 512
_NBLK = B // _BR


def _layers_body(adj_ref, h0_ref, w1_ref, w2_ref, out_ref,
                 abf_ref, h1_ref, rinv_ref):
    i = pl.program_id(0)

    @pl.when(i < _NBLK)
    def _():
        a = adj_ref[...]
        abf_ref[pl.ds(i * _BR, _BR), :] = a.astype(jnp.bfloat16)
        rinv = 1.0 / (jnp.sum(a, axis=-1, keepdims=True) + 1e-6)
        rinv_ref[pl.ds(i * _BR, _BR), :] = rinv
        m = jnp.dot(a, h0_ref[...], preferred_element_type=jnp.float32) * rinv
        z = jnp.maximum(
            jnp.dot(m, w1_ref[0], preferred_element_type=jnp.float32), 0.0)
        h1_ref[pl.ds(i * _BR, _BR), :] = h0_ref[pl.ds(i * _BR, _BR), :] + jnp.dot(
            z, w2_ref[0], preferred_element_type=jnp.float32)

    @pl.when(i >= _NBLK)
    def _():
        j = i - _NBLK
        a16 = abf_ref[pl.ds(j * _BR, _BR), :]
        m = jnp.dot(a16, h1_ref[...].astype(jnp.bfloat16),
                    preferred_element_type=jnp.float32)
        m = m * rinv_ref[pl.ds(j * _BR, _BR), :]
        z = jnp.maximum(
            jnp.dot(m, w1_ref[1], preferred_element_type=jnp.float32), 0.0)
        out_ref[...] = h1_ref[pl.ds(j * _BR, _BR), :] + jnp.dot(
            z, w2_ref[1], preferred_element_type=jnp.float32)


def _tc_layers(adj, h, w1, w2):
    return pl.pallas_call(
        _layers_body,
        out_shape=jax.ShapeDtypeStruct((B, ENT_DIM), jnp.float32),
        grid=(2 * _NBLK,),
        in_specs=[
            pl.BlockSpec((_BR, B), lambda i: (jnp.minimum(i, _NBLK - 1), 0)),
            pl.BlockSpec((B, ENT_DIM), lambda i: (0, 0)),
            pl.BlockSpec((NUM_LAYERS, ENT_DIM, HIDDEN_DIM), lambda i: (0, 0, 0)),
            pl.BlockSpec((NUM_LAYERS, HIDDEN_DIM, ENT_DIM), lambda i: (0, 0, 0)),
        ],
        out_specs=pl.BlockSpec(
            (_BR, ENT_DIM), lambda i: (jnp.maximum(i - _NBLK, 0), 0)),
        scratch_shapes=[
            pltpu.VMEM((B, B), jnp.bfloat16),
            pltpu.VMEM((B, ENT_DIM), jnp.float32),
            pltpu.VMEM((B, 1), jnp.float32),
        ],
        compiler_params=pltpu.CompilerParams(
            dimension_semantics=("arbitrary",),
            vmem_limit_bytes=62 * 1024 * 1024,
        ),
    )(adj, h, w1, w2)


def kernel(ent_relational_fearues, batch_ent_idxs, relational_adj_matrices,
           re_ratio, memory_cells, Win_W, gate_W, gate_b, layer_W1, layer_W2):
    idxs = batch_ent_idxs.astype(jnp.int32)

    n = B
    num_re = jnp.floor(n * jnp.asarray(re_ratio, dtype=jnp.float32)).astype(jnp.int32)
    perm = jax.random.permutation(jax.random.key(42), n)
    keep = jnp.where(jnp.arange(n) < num_re, 0.0, 1.0).astype(jnp.float32)
    row_mult = jnp.ones((n,), dtype=jnp.float32).at[perm].set(keep)
    mult = row_mult[:, None]

    sc_gather, sc_scatter, sc_copy = _make_sc_kernels()
    prev = sc_gather(memory_cells, idxs)

    gw2 = gate_W.reshape(2, ENT_DIM, ENT_DIM)
    memory_out = _tc_gate(ent_relational_fearues, Win_W, prev, gw2, gate_b, mult)

    copied = sc_copy(memory_cells)

    h = _tc_layers(relational_adj_matrices, memory_out, layer_W1, layer_W2)

    table_ref = jax.new_ref(copied)
    sc_scatter(memory_out, idxs, table_ref)
    new_memory_cells = jax.freeze(table_ref)
    return h, new_memory_cells
